# Initial kernel scaffold; baseline (speedup 1.0000x reference)
import jax
import jax.numpy as jnp
from jax.experimental import pallas as pl


def kernel(x, edge_index, batch, eps0, g0w1, g0b1, g0w2, g0b2, eps1, g1w1, g1b1, g1w2, g1b2, bn_gamma, bn_beta, p1w, p1b, p2w, p2b):
    raise NotImplementedError("write your pallas kernel here")


# speedup vs baseline: 5.8509x; 5.8509x over previous
import functools

import jax
import jax.numpy as jnp
from jax import lax
from jax.experimental import pallas as pl
from jax.experimental.pallas import tpu as pltpu
from jax.experimental.pallas import tpu_sc as plsc

N = 10000
E = 320000
D = 128
NHID = 256
NGRAPHS = 64

NC = 2
NS = 16
NW = NC * NS
EPW = E // NW
CH = 128
NFULL = EPW // CH
REM = EPW - NFULL * CH
RPT = 624
RPT_LAST = N - (NS - 1) * RPT

_sc_mesh = plsc.VectorSubcoreMesh(core_axis_name="c", subcore_axis_name="s")


@functools.partial(
    pl.kernel,
    mesh=_sc_mesh,
    out_type=jax.ShapeDtypeStruct((NC, N, D), jnp.float32),
    scratch_types=[
        pltpu.VMEM((CH,), jnp.int32),
        pltpu.VMEM((CH,), jnp.int32),
        pltpu.VMEM((CH, D), jnp.float32),
        pltpu.VMEM((REM,), jnp.int32),
        pltpu.VMEM((REM,), jnp.int32),
        pltpu.VMEM((REM, D), jnp.float32),
        pltpu.VMEM_SHARED((N, D), jnp.float32),
        pltpu.SemaphoreType.DMA,
    ],
)
def _sc_agg(h_hbm, src_hbm, dst_hbm, zeros_hbm, out_hbm,
            src_v, dst_v, rows_v, srcr_v, dstr_v, rowsr_v, acc, sem):
    c = lax.axis_index("c")
    s = lax.axis_index("s")
    wid = c * NS + s

    @pl.when(s < NS - 1)
    def _():
        pltpu.sync_copy(zeros_hbm.at[pl.ds(0, RPT), :],
                        acc.at[pl.ds(s * RPT, RPT), :])

    @pl.when(s == NS - 1)
    def _():
        pltpu.sync_copy(zeros_hbm, acc.at[pl.ds(s * RPT, RPT_LAST), :])

    plsc.subcore_barrier()

    base = wid * EPW

    def chunk(j, carry):
        e0 = base + j * CH
        pltpu.sync_copy(src_hbm.at[pl.ds(e0, CH)], src_v)
        pltpu.sync_copy(dst_hbm.at[pl.ds(e0, CH)], dst_v)
        pltpu.async_copy(h_hbm.at[src_v], rows_v, sem).wait()
        pltpu.sync_copy(rows_v, acc.at[dst_v], add=True)
        return carry

    lax.fori_loop(0, NFULL, chunk, 0)

    e0 = base + NFULL * CH
    pltpu.sync_copy(src_hbm.at[pl.ds(e0, REM)], srcr_v)
    pltpu.sync_copy(dst_hbm.at[pl.ds(e0, REM)], dstr_v)
    pltpu.async_copy(h_hbm.at[srcr_v], rowsr_v, sem).wait()
    pltpu.sync_copy(rowsr_v, acc.at[dstr_v], add=True)

    plsc.subcore_barrier()

    @pl.when(s < NS - 1)
    def _():
        pltpu.sync_copy(acc.at[pl.ds(s * RPT, RPT), :],
                        out_hbm.at[c, pl.ds(s * RPT, RPT), :])

    @pl.when(s == NS - 1)
    def _():
        pltpu.sync_copy(acc.at[pl.ds(s * RPT, RPT_LAST), :],
                        out_hbm.at[c, pl.ds(s * RPT, RPT_LAST), :])


def _mxu(a, b):
    return jnp.dot(a.astype(jnp.bfloat16), b.astype(jnp.bfloat16),
                   preferred_element_type=jnp.float32)


def _dense_body(x_ref, p_ref, eps_ref, w1_ref, b1_ref, w2_ref, b2_ref,
                gam_ref, bet_ref, out_ref):
    z = x_ref[...] * (1.0 + eps_ref[0, 0]) + p_ref[0] + p_ref[1]
    t = _mxu(z, w1_ref[...]) + b1_ref[...]
    t = jnp.maximum(t, 0.0)
    t = _mxu(t, w2_ref[...]) + b2_ref[...]
    t = jnp.maximum(t, 0.0)
    mean = jnp.mean(t, axis=0, keepdims=True)
    var = jnp.mean((t - mean) ** 2, axis=0, keepdims=True)
    out_ref[...] = (t - mean) * lax.rsqrt(var + 1e-5) * gam_ref[...] + bet_ref[...]


def _dense(x, p, eps, w1, b1, w2, b2, gam, bet):
    return pl.pallas_call(
        _dense_body,
        out_shape=jax.ShapeDtypeStruct((N, D), jnp.float32),
    )(x, p, eps, w1, b1, w2, b2, gam, bet)


def _final_body(h_ref, p_ref, eps_ref, w1_ref, b1_ref, w2_ref, b2_ref,
                gam_ref, bet_ref, batch_ref, p1w_ref, p1b_ref, p2w_ref,
                p2b_ref, out_ref):
    z = h_ref[...] * (1.0 + eps_ref[0, 0]) + p_ref[0] + p_ref[1]
    t = _mxu(z, w1_ref[...]) + b1_ref[...]
    t = jnp.maximum(t, 0.0)
    t = _mxu(t, w2_ref[...]) + b2_ref[...]
    t = jnp.maximum(t, 0.0)
    mean = jnp.mean(t, axis=0, keepdims=True)
    var = jnp.mean((t - mean) ** 2, axis=0, keepdims=True)
    hb = (t - mean) * lax.rsqrt(var + 1e-5) * gam_ref[...] + bet_ref[...]
    gids = lax.broadcasted_iota(jnp.int32, (N, NGRAPHS), 1)
    onehot = (batch_ref[...] == gids).astype(jnp.float32)
    pooled = lax.dot_general(onehot, hb, (((0,), (0,)), ((), ())),
                             preferred_element_type=jnp.float32, precision=lax.Precision.HIGHEST)
    ones = jnp.ones((N, 1), dtype=jnp.float32)
    counts = lax.dot_general(onehot, ones, (((0,), (0,)), ((), ())),
                             preferred_element_type=jnp.float32, precision=lax.Precision.HIGHEST)
    pooled = pooled / jnp.maximum(counts, 1.0)
    o = jnp.maximum(_mxu(pooled, p1w_ref[...]) + p1b_ref[...], 0.0)
    out_ref[...] = _mxu(o, p2w_ref[...]) + p2b_ref[...]


def _final(h, p, eps, w1, b1, w2, b2, gam, bet, batch2d, p1w, p1b, p2w, p2b):
    return pl.pallas_call(
        _final_body,
        out_shape=jax.ShapeDtypeStruct((NGRAPHS, D), jnp.float32),
    )(h, p, eps, w1, b1, w2, b2, gam, bet, batch2d, p1w, p1b, p2w, p2b)


def kernel(x, edge_index, batch, eps0, g0w1, g0b1, g0w2, g0b2,
           eps1, g1w1, g1b1, g1w2, g1b2, bn_gamma, bn_beta,
           p1w, p1b, p2w, p2b):
    src = edge_index[0]
    dst = edge_index[1]
    zeros = jnp.zeros((RPT_LAST, D), jnp.float32)
    eps0_2d = eps0.reshape(1, 1)
    eps1_2d = eps1.reshape(1, 1)
    batch2d = batch.reshape(N, 1)
    gam = bn_gamma.reshape(1, D)
    bet = bn_beta.reshape(1, D)

    p0 = _sc_agg(x, src, dst, zeros)
    h0 = _dense(x, p0, eps0_2d, g0w1, g0b1.reshape(1, -1),
                g0w2, g0b2.reshape(1, -1), gam, bet)
    p1_ = _sc_agg(h0, src, dst, zeros)
    out = _final(h0, p1_, eps1_2d, g1w1, g1b1.reshape(1, -1),
                 g1w2, g1b2.reshape(1, -1), gam, bet, batch2d,
                 p1w, p1b.reshape(1, -1), p2w, p2b.reshape(1, -1))
    return out


# speedup vs baseline: 11.3018x; 1.9316x over previous
import functools

import jax
import jax.numpy as jnp
from jax import lax
from jax.experimental import pallas as pl
from jax.experimental.pallas import tpu as pltpu
from jax.experimental.pallas import tpu_sc as plsc

N = 10000
E = 320000
D = 128
NHID = 256
NGRAPHS = 64

NC = 2
NS = 16
NW = NC * NS
EPW = E // NW
CH = 128
NFULL = EPW // CH
REM = EPW - NFULL * CH
RPT = 624
RPT_LAST = N - (NS - 1) * RPT

_sc_mesh = plsc.VectorSubcoreMesh(core_axis_name="c", subcore_axis_name="s")


@functools.partial(
    pl.kernel,
    mesh=_sc_mesh,
    out_type=jax.ShapeDtypeStruct((NC, N, D), jnp.float32),
    scratch_types=[
        pltpu.VMEM((NFULL, CH), jnp.int32),
        pltpu.VMEM((CH,), jnp.int32),
        pltpu.VMEM((CH,), jnp.int32),
        pltpu.VMEM((CH, D), jnp.float32),
        pltpu.VMEM((CH, D), jnp.float32),
        pltpu.VMEM((REM,), jnp.int32),
        pltpu.VMEM((REM,), jnp.int32),
        pltpu.VMEM_SHARED((N, D), jnp.float32),
        pltpu.SemaphoreType.DMA,
        pltpu.SemaphoreType.DMA,
        pltpu.SemaphoreType.DMA,
        pltpu.SemaphoreType.DMA,
    ],
)
def _sc_agg(h_hbm, srcm_hbm, dstm_hbm, srcr_hbm, dstr_hbm, zeros_hbm, out_hbm,
            src_v, dst0, dst1, rows0, rows1, srcr_v, dstr_v, acc,
            semg0, semg1, semd0, semd1):
    c = lax.axis_index("c")
    s = lax.axis_index("s")
    wid = c * NS + s

    @pl.when(s < NS - 1)
    def _():
        pltpu.sync_copy(zeros_hbm.at[pl.ds(0, RPT), :],
                        acc.at[pl.ds(s * RPT, RPT), :])

    @pl.when(s == NS - 1)
    def _():
        pltpu.sync_copy(zeros_hbm, acc.at[pl.ds(s * RPT, RPT_LAST), :])

    pltpu.sync_copy(srcm_hbm.at[wid], src_v)
    pltpu.sync_copy(srcr_hbm.at[wid], srcr_v)
    pltpu.sync_copy(dstr_hbm.at[wid], dstr_v)
    plsc.subcore_barrier()

    rbufs = (rows0, rows1)
    gsems = (semg0, semg1)
    dbufs = (dst0, dst1)
    dsems = (semd0, semd1)

    pltpu.async_copy(h_hbm.at[src_v.at[0]], rows0, semg0)
    pltpu.async_copy(dstm_hbm.at[wid, 0], dst0, semd0)

    def body(j0, carry):
        for b in range(2):
            j = j0 * 2 + b
            nxt = j + 1
            nb = (b + 1) % 2
            @pl.when(nxt < NFULL)
            def _():
                pltpu.async_copy(h_hbm.at[src_v.at[nxt]], rbufs[nb],
                                 gsems[nb])
                pltpu.async_copy(dstm_hbm.at[wid, nxt], dbufs[nb], dsems[nb])
            pltpu.make_async_copy(h_hbm.at[src_v.at[j]], rbufs[b],
                                  gsems[b]).wait()
            pltpu.make_async_copy(dstm_hbm.at[wid, j], dbufs[b],
                                  dsems[b]).wait()
            pltpu.sync_copy(rbufs[b], acc.at[dbufs[b]], add=True)
        return carry

    lax.fori_loop(0, NFULL // 2, body, 0)

    pltpu.async_copy(h_hbm.at[srcr_v], rows0.at[pl.ds(0, REM), :],
                     semg0).wait()
    pltpu.sync_copy(rows0.at[pl.ds(0, REM), :], acc.at[dstr_v], add=True)

    plsc.subcore_barrier()

    @pl.when(s < NS - 1)
    def _():
        pltpu.sync_copy(acc.at[pl.ds(s * RPT, RPT), :],
                        out_hbm.at[c, pl.ds(s * RPT, RPT), :])

    @pl.when(s == NS - 1)
    def _():
        pltpu.sync_copy(acc.at[pl.ds(s * RPT, RPT_LAST), :],
                        out_hbm.at[c, pl.ds(s * RPT, RPT_LAST), :])


def _mxu(a, b):
    return jnp.dot(a.astype(jnp.bfloat16), b.astype(jnp.bfloat16),
                   preferred_element_type=jnp.float32)


def _dense_body(x_ref, p_ref, eps_ref, w1_ref, b1_ref, w2_ref, b2_ref,
                gam_ref, bet_ref, out_ref):
    z = x_ref[...] * (1.0 + eps_ref[0, 0]) + p_ref[0] + p_ref[1]
    t = _mxu(z, w1_ref[...]) + b1_ref[...]
    t = jnp.maximum(t, 0.0)
    t = _mxu(t, w2_ref[...]) + b2_ref[...]
    t = jnp.maximum(t, 0.0)
    mean = jnp.mean(t, axis=0, keepdims=True)
    var = jnp.mean((t - mean) ** 2, axis=0, keepdims=True)
    out_ref[...] = (t - mean) * lax.rsqrt(var + 1e-5) * gam_ref[...] + bet_ref[...]


def _dense(x, p, eps, w1, b1, w2, b2, gam, bet):
    return pl.pallas_call(
        _dense_body,
        out_shape=jax.ShapeDtypeStruct((N, D), jnp.float32),
    )(x, p, eps, w1, b1, w2, b2, gam, bet)


def _final_body(h_ref, p_ref, eps_ref, w1_ref, b1_ref, w2_ref, b2_ref,
                gam_ref, bet_ref, batch_ref, p1w_ref, p1b_ref, p2w_ref,
                p2b_ref, out_ref):
    z = h_ref[...] * (1.0 + eps_ref[0, 0]) + p_ref[0] + p_ref[1]
    t = _mxu(z, w1_ref[...]) + b1_ref[...]
    t = jnp.maximum(t, 0.0)
    t = _mxu(t, w2_ref[...]) + b2_ref[...]
    t = jnp.maximum(t, 0.0)
    mean = jnp.mean(t, axis=0, keepdims=True)
    var = jnp.mean((t - mean) ** 2, axis=0, keepdims=True)
    hb = (t - mean) * lax.rsqrt(var + 1e-5) * gam_ref[...] + bet_ref[...]
    gids = lax.broadcasted_iota(jnp.int32, (N, NGRAPHS), 1)
    onehot = (batch_ref[...] == gids).astype(jnp.float32)
    pooled = lax.dot_general(onehot, hb, (((0,), (0,)), ((), ())),
                             preferred_element_type=jnp.float32, precision=lax.Precision.HIGHEST)
    ones = jnp.ones((N, 1), dtype=jnp.float32)
    counts = lax.dot_general(onehot, ones, (((0,), (0,)), ((), ())),
                             preferred_element_type=jnp.float32, precision=lax.Precision.HIGHEST)
    pooled = pooled / jnp.maximum(counts, 1.0)
    o = jnp.maximum(_mxu(pooled, p1w_ref[...]) + p1b_ref[...], 0.0)
    out_ref[...] = _mxu(o, p2w_ref[...]) + p2b_ref[...]


def _final(h, p, eps, w1, b1, w2, b2, gam, bet, batch2d, p1w, p1b, p2w, p2b):
    return pl.pallas_call(
        _final_body,
        out_shape=jax.ShapeDtypeStruct((NGRAPHS, D), jnp.float32),
    )(h, p, eps, w1, b1, w2, b2, gam, bet, batch2d, p1w, p1b, p2w, p2b)


def kernel(x, edge_index, batch, eps0, g0w1, g0b1, g0w2, g0b2,
           eps1, g1w1, g1b1, g1w2, g1b2, bn_gamma, bn_beta,
           p1w, p1b, p2w, p2b):
    src = edge_index[0].reshape(NW, EPW)
    dst = edge_index[1].reshape(NW, EPW)
    srcm = src[:, :NFULL * CH].reshape(NW, NFULL, CH)
    dstm = dst[:, :NFULL * CH].reshape(NW, NFULL, CH)
    srcr = src[:, NFULL * CH:]
    dstr = dst[:, NFULL * CH:]
    zeros = jnp.zeros((RPT_LAST, D), jnp.float32)
    eps0_2d = eps0.reshape(1, 1)
    eps1_2d = eps1.reshape(1, 1)
    batch2d = batch.reshape(N, 1)
    gam = bn_gamma.reshape(1, D)
    bet = bn_beta.reshape(1, D)

    p0 = _sc_agg(x, srcm, dstm, srcr, dstr, zeros)
    h0 = _dense(x, p0, eps0_2d, g0w1, g0b1.reshape(1, -1),
                g0w2, g0b2.reshape(1, -1), gam, bet)
    p1_ = _sc_agg(h0, srcm, dstm, srcr, dstr, zeros)
    out = _final(h0, p1_, eps1_2d, g1w1, g1b1.reshape(1, -1),
                 g1w2, g1b2.reshape(1, -1), gam, bet, batch2d,
                 p1w, p1b.reshape(1, -1), p2w, p2b.reshape(1, -1))
    return out


# speedup vs baseline: 12.3594x; 1.0936x over previous
import functools

import jax
import jax.numpy as jnp
from jax import lax
from jax.experimental import pallas as pl
from jax.experimental.pallas import tpu as pltpu
from jax.experimental.pallas import tpu_sc as plsc

N = 10000
E = 320000
D = 128
NHID = 256
NGRAPHS = 64

NC = 2
NS = 16
NW = NC * NS
EPW = E // NW
CH = 128
NFULL = EPW // CH
REM = EPW - NFULL * CH
RPT = 624
RPT_LAST = N - (NS - 1) * RPT

_sc_mesh = plsc.VectorSubcoreMesh(core_axis_name="c", subcore_axis_name="s")


@functools.partial(
    pl.kernel,
    mesh=_sc_mesh,
    out_type=jax.ShapeDtypeStruct((NC, N, D), jnp.float32),
    scratch_types=[
        pltpu.VMEM((NFULL * CH,), jnp.int32),
        pltpu.VMEM((CH,), jnp.int32),
        pltpu.VMEM((CH,), jnp.int32),
        pltpu.VMEM((CH, D), jnp.float32),
        pltpu.VMEM((CH, D), jnp.float32),
        pltpu.VMEM((REM,), jnp.int32),
        pltpu.VMEM((REM,), jnp.int32),
        pltpu.VMEM_SHARED((N, D), jnp.float32),
        pltpu.SemaphoreType.DMA,
        pltpu.SemaphoreType.DMA,
        pltpu.SemaphoreType.DMA,
        pltpu.SemaphoreType.DMA,
    ],
)
def _sc_agg(h_hbm, edge_hbm, zeros_hbm, out_hbm,
            src_v, dst0, dst1, rows0, rows1, srcr_v, dstr_v, acc,
            semg0, semg1, semd0, semd1):
    c = lax.axis_index("c")
    s = lax.axis_index("s")
    wid = c * NS + s
    ebase = wid * EPW
    dbase = E + wid * EPW

    pltpu.async_copy(edge_hbm.at[pl.ds(ebase, NFULL * CH)], src_v, semg0)
    pltpu.async_copy(edge_hbm.at[pl.ds(ebase + NFULL * CH, REM)], srcr_v,
                     semg1)
    pltpu.async_copy(edge_hbm.at[pl.ds(dbase + NFULL * CH, REM)], dstr_v,
                     semd1)

    @pl.when(s < NS - 1)
    def _():
        pltpu.sync_copy(zeros_hbm.at[pl.ds(0, RPT), :],
                        acc.at[pl.ds(s * RPT, RPT), :])

    @pl.when(s == NS - 1)
    def _():
        pltpu.sync_copy(zeros_hbm, acc.at[pl.ds(s * RPT, RPT_LAST), :])

    pltpu.make_async_copy(edge_hbm.at[pl.ds(ebase, NFULL * CH)], src_v,
                          semg0).wait()
    pltpu.make_async_copy(edge_hbm.at[pl.ds(ebase + NFULL * CH, REM)],
                          srcr_v, semg1).wait()
    pltpu.make_async_copy(edge_hbm.at[pl.ds(dbase + NFULL * CH, REM)],
                          dstr_v, semd1).wait()

    rbufs = (rows0, rows1)
    gsems = (semg0, semg1)
    dbufs = (dst0, dst1)
    dsems = (semd0, semd1)

    def _srcidx(j):
        return src_v.at[pl.ds(j * CH, CH)]

    pltpu.async_copy(h_hbm.at[_srcidx(0)], rows0, semg0)
    pltpu.async_copy(edge_hbm.at[pl.ds(dbase, CH)], dst0, semd0)
    plsc.subcore_barrier()

    def body(j0, carry):
        for b in range(2):
            j = j0 * 2 + b
            nxt = j + 1
            nb = (b + 1) % 2

            @pl.when(nxt < NFULL)
            def _():
                pltpu.async_copy(h_hbm.at[_srcidx(nxt)], rbufs[nb],
                                 gsems[nb])
                pltpu.async_copy(edge_hbm.at[pl.ds(dbase + nxt * CH, CH)],
                                 dbufs[nb], dsems[nb])

            pltpu.make_async_copy(h_hbm.at[_srcidx(j)], rbufs[b],
                                  gsems[b]).wait()
            pltpu.make_async_copy(edge_hbm.at[pl.ds(dbase + j * CH, CH)],
                                  dbufs[b], dsems[b]).wait()
            pltpu.sync_copy(rbufs[b], acc.at[dbufs[b]], add=True)
        return carry

    lax.fori_loop(0, NFULL // 2, body, 0)

    pltpu.async_copy(h_hbm.at[srcr_v], rows0.at[pl.ds(0, REM), :],
                     semg0).wait()
    pltpu.sync_copy(rows0.at[pl.ds(0, REM), :], acc.at[dstr_v], add=True)

    plsc.subcore_barrier()

    @pl.when(s < NS - 1)
    def _():
        pltpu.sync_copy(acc.at[pl.ds(s * RPT, RPT), :],
                        out_hbm.at[c, pl.ds(s * RPT, RPT), :])

    @pl.when(s == NS - 1)
    def _():
        pltpu.sync_copy(acc.at[pl.ds(s * RPT, RPT_LAST), :],
                        out_hbm.at[c, pl.ds(s * RPT, RPT_LAST), :])


def _mxu(a, b):
    return jnp.dot(a.astype(jnp.bfloat16), b.astype(jnp.bfloat16),
                   preferred_element_type=jnp.float32)


def _dense_body(x_ref, p_ref, eps_ref, w1_ref, b1_ref, w2_ref, b2_ref,
                gam_ref, bet_ref, out_ref):
    z = x_ref[...] * (1.0 + eps_ref[0, 0]) + p_ref[0] + p_ref[1]
    t = _mxu(z, w1_ref[...]) + b1_ref[...]
    t = jnp.maximum(t, 0.0)
    t = _mxu(t, w2_ref[...]) + b2_ref[...]
    t = jnp.maximum(t, 0.0)
    mean = jnp.mean(t, axis=0, keepdims=True)
    var = jnp.mean((t - mean) ** 2, axis=0, keepdims=True)
    out_ref[...] = (t - mean) * lax.rsqrt(var + 1e-5) * gam_ref[...] + bet_ref[...]


def _dense(x, p, eps, w1, b1, w2, b2, gam, bet):
    return pl.pallas_call(
        _dense_body,
        out_shape=jax.ShapeDtypeStruct((N, D), jnp.float32),
    )(x, p, eps, w1, b1, w2, b2, gam, bet)


def _final_body(h_ref, p_ref, eps_ref, w1_ref, b1_ref, w2_ref, b2_ref,
                gam_ref, bet_ref, batch_ref, p1w_ref, p1b_ref, p2w_ref,
                p2b_ref, out_ref):
    z = h_ref[...] * (1.0 + eps_ref[0, 0]) + p_ref[0] + p_ref[1]
    t = _mxu(z, w1_ref[...]) + b1_ref[...]
    t = jnp.maximum(t, 0.0)
    t = _mxu(t, w2_ref[...]) + b2_ref[...]
    t = jnp.maximum(t, 0.0)
    mean = jnp.mean(t, axis=0, keepdims=True)
    var = jnp.mean((t - mean) ** 2, axis=0, keepdims=True)
    hb = (t - mean) * lax.rsqrt(var + 1e-5) * gam_ref[...] + bet_ref[...]
    gids = lax.broadcasted_iota(jnp.int32, (NGRAPHS, N), 0)
    onehot = (batch_ref[...] == gids).astype(jnp.float32)
    pooled = jnp.dot(onehot, hb, preferred_element_type=jnp.float32,
                     precision=lax.Precision.HIGHEST)
    counts = jnp.sum(onehot, axis=1, keepdims=True)
    pooled = pooled / jnp.maximum(counts, 1.0)
    o = jnp.maximum(_mxu(pooled, p1w_ref[...]) + p1b_ref[...], 0.0)
    out_ref[...] = _mxu(o, p2w_ref[...]) + p2b_ref[...]


def _final(h, p, eps, w1, b1, w2, b2, gam, bet, batch2d, p1w, p1b, p2w, p2b):
    return pl.pallas_call(
        _final_body,
        out_shape=jax.ShapeDtypeStruct((NGRAPHS, D), jnp.float32),
    )(h, p, eps, w1, b1, w2, b2, gam, bet, batch2d, p1w, p1b, p2w, p2b)


def kernel(x, edge_index, batch, eps0, g0w1, g0b1, g0w2, g0b2,
           eps1, g1w1, g1b1, g1w2, g1b2, bn_gamma, bn_beta,
           p1w, p1b, p2w, p2b):
    edges = edge_index.reshape(2 * E)
    zeros = jnp.zeros((RPT_LAST, D), jnp.float32)
    eps0_2d = eps0.reshape(1, 1)
    eps1_2d = eps1.reshape(1, 1)
    batch2d = batch.reshape(1, N)
    gam = bn_gamma.reshape(1, D)
    bet = bn_beta.reshape(1, D)

    p0 = _sc_agg(x, edges, zeros)
    h0 = _dense(x, p0, eps0_2d, g0w1, g0b1.reshape(1, -1),
                g0w2, g0b2.reshape(1, -1), gam, bet)
    p1_ = _sc_agg(h0, edges, zeros)
    out = _final(h0, p1_, eps1_2d, g1w1, g1b1.reshape(1, -1),
                 g1w2, g1b2.reshape(1, -1), gam, bet, batch2d,
                 p1w, p1b.reshape(1, -1), p2w, p2b.reshape(1, -1))
    return out
